# Initial kernel scaffold; baseline (speedup 1.0000x reference)
import jax
import jax.numpy as jnp
from jax.experimental import pallas as pl


def kernel(input, W_mlp, b_mlp, W_gcn, b_gcn):
    raise NotImplementedError("write your pallas kernel here")


# speedup vs baseline: 4.7507x; 4.7507x over previous
import functools

import jax
import jax.numpy as jnp
from jax import lax
from jax.experimental import pallas as pl

IN_F = 16
OUT_F = 32
NUM_PT = 256
K = 16
UP_FTR = 2
B = 8
N = NUM_PT * UP_FTR
D_IN = NUM_PT * IN_F
D_OUT = NUM_PT * UP_FTR * OUT_F

MLP_BLK = 512
BISECT_ITERS = 28


def _mlp_kernel(x_ref, w_ref, b_ref, out_ref):
    acc = lax.dot_general(
        x_ref[...].astype(jnp.bfloat16), w_ref[...].astype(jnp.bfloat16),
        dimension_numbers=(((1,), (1,)), ((), ())),
        preferred_element_type=jnp.float32,
    )
    out_ref[...] = acc + b_ref[...]


def _gcn_kernel(feat_ref, wt_ref, bg_ref, out_ref):
    feat = feat_ref[0, :, :]
    g = lax.dot_general(
        feat, feat, dimension_numbers=(((0,), (0,)), ((), ())),
        preferred_element_type=jnp.float32,
        precision=lax.Precision.HIGHEST,
    )
    sq = feat * feat
    n_row = jnp.sum(sq, axis=0, keepdims=True)
    ones = jnp.ones((OUT_F, 1), dtype=jnp.float32)
    n_col = lax.dot_general(
        sq, ones, dimension_numbers=(((0,), (0,)), ((), ())),
        preferred_element_type=jnp.float32,
        precision=lax.Precision.HIGHEST,
    )
    dist = n_col + n_row - 2.0 * g

    lo = jnp.min(dist, axis=0, keepdims=True) - 1e-3
    hi = jnp.max(dist, axis=0, keepdims=True) + 1e-3

    def body(_, carry):
        lo_, hi_ = carry
        mid = 0.5 * (lo_ + hi_)
        cnt = jnp.sum((dist <= mid).astype(jnp.float32), axis=0, keepdims=True)
        pred = cnt >= float(K)
        return (jnp.where(pred, lo_, mid), jnp.where(pred, mid, hi_))

    lo, hi = lax.fori_loop(0, BISECT_ITERS, body, (lo, hi))
    adj_t = jnp.where(dist <= hi, 1.0 / K, 0.0)

    support_t = lax.dot_general(
        wt_ref[...], feat, dimension_numbers=(((1,), (0,)), ((), ())),
        preferred_element_type=jnp.float32,
        precision=lax.Precision.HIGHEST,
    )
    agg = lax.dot_general(
        support_t, adj_t, dimension_numbers=(((1,), (0,)), ((), ())),
        preferred_element_type=jnp.float32,
        precision=lax.Precision.HIGHEST,
    )
    out_ref[0, :, :] = agg + bg_ref[...]


@jax.jit
def kernel(input, W_mlp, b_mlp, W_gcn, b_gcn):
    x = input.astype(jnp.float32)
    num_blk = D_OUT // MLP_BLK
    b2 = b_mlp.reshape(1, D_OUT)
    feat_flat = pl.pallas_call(
        _mlp_kernel,
        grid=(num_blk,),
        in_specs=[
            pl.BlockSpec((B, D_IN), lambda j: (0, 0)),
            pl.BlockSpec((MLP_BLK, D_IN), lambda j: (j, 0)),
            pl.BlockSpec((1, MLP_BLK), lambda j: (0, j)),
        ],
        out_specs=pl.BlockSpec((B, MLP_BLK), lambda j: (0, j)),
        out_shape=jax.ShapeDtypeStruct((B, D_OUT), jnp.float32),
    )(x, W_mlp, b2)
    feat = feat_flat.reshape(B, OUT_F, N)

    wt = W_gcn[0].T
    bg = b_gcn[0]
    out = pl.pallas_call(
        _gcn_kernel,
        grid=(B,),
        in_specs=[
            pl.BlockSpec((1, OUT_F, N), lambda b: (b, 0, 0)),
            pl.BlockSpec((OUT_F, OUT_F), lambda b: (0, 0)),
            pl.BlockSpec((OUT_F, 1), lambda b: (0, 0)),
        ],
        out_specs=pl.BlockSpec((1, OUT_F, N), lambda b: (b, 0, 0)),
        out_shape=jax.ShapeDtypeStruct((B, OUT_F, N), jnp.float32),
    )(feat, wt, bg)

    return out.reshape(B, D_OUT)
